# baseline (device time: 55764 ns/iter reference)
import jax
import jax.numpy as jnp
from jax import lax
from jax.experimental import pallas as pl
from jax.experimental.pallas import tpu as pltpu

N_DEV = 16
B, SQ, D = 1, 512, 1024
HQ, HKV, DH = 8, 2, 128
GROUP = HQ // HKV
SCALE = 0.08838834764831843
CHUNK = SQ // N_DEV
NB = 4
BR = SQ // NB
PPB = N_DEV // NB


def kernel(x, Wq, Wo, K_ext, V_ext):
    skv = K_ext.shape[1]
    x2 = x.reshape(SQ, D).astype(jnp.bfloat16)
    k2 = K_ext.reshape(skv, HKV * DH).astype(jnp.bfloat16)
    v2 = V_ext.reshape(skv, HKV * DH).astype(jnp.bfloat16)
    wq = Wq.astype(jnp.bfloat16)
    wo = Wo.astype(jnp.bfloat16)

    def body(x_ref, wq_ref, wo_ref, k_ref, v_ref, out_ref,
             o_snd, l_scr, rs_o, rs_l, y_div, y_snd, ag_ref,
             o_ssem, o_rsem, l_ssem, l_rsem, g_ssem, g_rsem):
        my = lax.axis_index("i")

        barrier = pltpu.get_barrier_semaphore()
        for k in range(1, N_DEV):
            pl.semaphore_signal(barrier, inc=1,
                                device_id=((my + k) % N_DEV,),
                                device_id_type=pl.DeviceIdType.MESH)
        pl.semaphore_wait(barrier, N_DEV - 1)

        q = jnp.dot(x_ref[...], wq_ref[...],
                    preferred_element_type=jnp.float32).astype(jnp.bfloat16)

        rd_o = [None] * N_DEV
        rd_l = [None] * N_DEV
        for b in range(NB):
            r0 = b * BR
            for h in range(HQ):
                g = h // GROUP
                qh = q[r0:r0 + BR, h * DH:(h + 1) * DH]
                kh = k_ref[:, g * DH:(g + 1) * DH]
                vh = v_ref[:, g * DH:(g + 1) * DH]
                s = lax.dot_general(qh, kh, (((1,), (1,)), ((), ())),
                                    preferred_element_type=jnp.float32)
                p = jnp.exp(s * SCALE).astype(jnp.bfloat16)
                l_scr[r0:r0 + BR, h:h + 1] = jnp.sum(
                    p, axis=1, keepdims=True, dtype=jnp.float32)
                o_snd[r0:r0 + BR, h * DH:(h + 1) * DH] = jnp.dot(
                    p, vh, preferred_element_type=jnp.float32
                ).astype(jnp.bfloat16)
            for r in range(PPB):
                pid = b * PPB + r
                rd_o[pid] = pltpu.make_async_remote_copy(
                    src_ref=o_snd.at[pl.ds(pid * CHUNK, CHUNK), :],
                    dst_ref=rs_o.at[my],
                    send_sem=o_ssem.at[pid], recv_sem=o_rsem.at[my],
                    device_id=(pid,), device_id_type=pl.DeviceIdType.MESH)
                rd_l[pid] = pltpu.make_async_remote_copy(
                    src_ref=l_scr.at[pl.ds(pid * CHUNK, CHUNK), :],
                    dst_ref=rs_l.at[my],
                    send_sem=l_ssem.at[pid], recv_sem=l_rsem.at[my],
                    device_id=(pid,), device_id_type=pl.DeviceIdType.MESH)

                @pl.when(pid != my)
                def _(pid=pid):
                    rd_o[pid].start()
                    rd_l[pid].start()

        o_acc = o_snd[pl.ds(my * CHUNK, CHUNK), :].astype(jnp.float32)
        l_acc = l_scr[pl.ds(my * CHUNK, CHUNK), :]
        for s_ in range(N_DEV):
            rro = pltpu.make_async_remote_copy(
                src_ref=rs_o.at[s_], dst_ref=rs_o.at[s_],
                send_sem=o_rsem.at[s_], recv_sem=o_rsem.at[s_],
                device_id=(0,), device_id_type=pl.DeviceIdType.MESH)
            rrl = pltpu.make_async_remote_copy(
                src_ref=rs_l.at[s_], dst_ref=rs_l.at[s_],
                send_sem=l_rsem.at[s_], recv_sem=l_rsem.at[s_],
                device_id=(0,), device_id_type=pl.DeviceIdType.MESH)

            @pl.when(s_ != my)
            def _(rro=rro, rrl=rrl):
                rro.wait_recv()
                rrl.wait_recv()

            skip = s_ == my
            o_acc = o_acc + jnp.where(skip, 0.0, rs_o[s_].astype(jnp.float32))
            l_acc = l_acc + jnp.where(skip, 0.0, rs_l[s_])

        for h in range(HQ):
            y_div[:, h * DH:(h + 1) * DH] = (
                o_acc[:, h * DH:(h + 1) * DH] / l_acc[:, h:h + 1])
        y = jnp.dot(y_div[...].astype(jnp.bfloat16), wo_ref[...],
                    preferred_element_type=jnp.float32)
        out_ref[pl.ds(my * CHUNK, CHUNK), :] = y
        y_snd[...] = y.astype(jnp.bfloat16)

        rd_g = [None] * N_DEV
        for pid in range(N_DEV):
            rd_g[pid] = pltpu.make_async_remote_copy(
                src_ref=y_snd, dst_ref=ag_ref.at[my],
                send_sem=g_ssem.at[pid], recv_sem=g_rsem.at[my],
                device_id=(pid,), device_id_type=pl.DeviceIdType.MESH)

            @pl.when(pid != my)
            def _(pid=pid):
                rd_g[pid].start()

        for s_ in range(N_DEV):
            rrg = pltpu.make_async_remote_copy(
                src_ref=ag_ref.at[s_], dst_ref=ag_ref.at[s_],
                send_sem=g_rsem.at[s_], recv_sem=g_rsem.at[s_],
                device_id=(0,), device_id_type=pl.DeviceIdType.MESH)

            @pl.when(s_ != my)
            def _(rrg=rrg, s_=s_):
                rrg.wait_recv()
                out_ref[pl.ds(s_ * CHUNK, CHUNK), :] = (
                    ag_ref[s_].astype(jnp.float32))

        for pid in range(N_DEV):
            @pl.when(pid != my)
            def _(pid=pid):
                rd_o[pid].wait_send()
                rd_l[pid].wait_send()
                rd_g[pid].wait_send()

    out = pl.pallas_call(
        body,
        out_shape=jax.ShapeDtypeStruct((SQ, D), jnp.float32),
        in_specs=[pl.BlockSpec(memory_space=pltpu.VMEM)] * 5,
        out_specs=pl.BlockSpec(memory_space=pltpu.VMEM),
        scratch_shapes=[
            pltpu.VMEM((SQ, D), jnp.bfloat16),
            pltpu.VMEM((SQ, HQ), jnp.float32),
            pltpu.VMEM((N_DEV, CHUNK, D), jnp.bfloat16),
            pltpu.VMEM((N_DEV, CHUNK, HQ), jnp.float32),
            pltpu.VMEM((CHUNK, D), jnp.float32),
            pltpu.VMEM((CHUNK, D), jnp.bfloat16),
            pltpu.VMEM((N_DEV, CHUNK, D), jnp.bfloat16),
            pltpu.SemaphoreType.DMA((N_DEV,)),
            pltpu.SemaphoreType.DMA((N_DEV,)),
            pltpu.SemaphoreType.DMA((N_DEV,)),
            pltpu.SemaphoreType.DMA((N_DEV,)),
            pltpu.SemaphoreType.DMA((N_DEV,)),
            pltpu.SemaphoreType.DMA((N_DEV,)),
        ],
        compiler_params=pltpu.CompilerParams(collective_id=0),
    )(x2, wq, wo, k2, v2)
    return out.reshape(B, SQ, D)


# device time: 46878 ns/iter; 1.1896x vs baseline; 1.1896x over previous
import jax
import jax.numpy as jnp
from jax import lax
from jax.experimental import pallas as pl
from jax.experimental.pallas import tpu as pltpu

N_DEV = 16
B, SQ, D = 1, 512, 1024
HQ, HKV, DH = 8, 2, 128
GROUP = HQ // HKV
SCALE = 0.08838834764831843
CHUNK = SQ // N_DEV


def kernel(x, Wq, Wo, K_ext, V_ext):
    skv = K_ext.shape[1]
    x2 = x.reshape(SQ, D)
    k2 = K_ext.reshape(skv, HKV * DH)
    v2 = V_ext.reshape(skv, HKV * DH)

    def body(x_ref, wq_ref, wo_ref, k_ref, v_ref, out_ref,
             o_snd, l_scr, rs_o, rs_l, y_div, y_snd, ag_ref,
             o_ssem, o_rsem, l_ssem, l_rsem, g_ssem, g_rsem):
        my = lax.axis_index("i")

        barrier = pltpu.get_barrier_semaphore()
        for k in range(1, N_DEV):
            pl.semaphore_signal(barrier, inc=1,
                                device_id=((my + k) % N_DEV,),
                                device_id_type=pl.DeviceIdType.MESH)
        pl.semaphore_wait(barrier, N_DEV - 1)

        q = jnp.dot(x_ref[...].astype(jnp.bfloat16),
                    wq_ref[...].astype(jnp.bfloat16),
                    preferred_element_type=jnp.float32).astype(jnp.bfloat16)

        rd_o = [[None] * HQ for _ in range(N_DEV)]
        rd_l = [None] * N_DEV
        for h in range(HQ):
            g = h // GROUP
            qh = q[:, h * DH:(h + 1) * DH]
            kh = k_ref[:, g * DH:(g + 1) * DH].astype(jnp.bfloat16)
            vh = v_ref[:, g * DH:(g + 1) * DH].astype(jnp.bfloat16)
            s = lax.dot_general(qh, kh, (((1,), (1,)), ((), ())),
                                preferred_element_type=jnp.float32)
            p = jnp.exp(s * SCALE).astype(jnp.bfloat16)
            l_scr[:, h:h + 1] = jnp.sum(
                p, axis=1, keepdims=True, dtype=jnp.float32)
            o_snd[:, h * DH:(h + 1) * DH] = jnp.dot(
                p, vh, preferred_element_type=jnp.float32
            ).astype(jnp.bfloat16)
            for pid in range(N_DEV):
                rd_o[pid][h] = pltpu.make_async_remote_copy(
                    src_ref=o_snd.at[pl.ds(pid * CHUNK, CHUNK),
                                     h * DH:(h + 1) * DH],
                    dst_ref=rs_o.at[my, :, h * DH:(h + 1) * DH],
                    send_sem=o_ssem.at[pid, h], recv_sem=o_rsem.at[my, h],
                    device_id=(pid,), device_id_type=pl.DeviceIdType.MESH)

                @pl.when(pid != my)
                def _(pid=pid, h=h):
                    rd_o[pid][h].start()

        for pid in range(N_DEV):
            rd_l[pid] = pltpu.make_async_remote_copy(
                src_ref=l_scr.at[pl.ds(pid * CHUNK, CHUNK), :],
                dst_ref=rs_l.at[my],
                send_sem=l_ssem.at[pid], recv_sem=l_rsem.at[my],
                device_id=(pid,), device_id_type=pl.DeviceIdType.MESH)

            @pl.when(pid != my)
            def _(pid=pid):
                rd_l[pid].start()

        o_acc = o_snd[pl.ds(my * CHUNK, CHUNK), :].astype(jnp.float32)
        l_acc = l_scr[pl.ds(my * CHUNK, CHUNK), :]
        for s_ in range(N_DEV):
            for h in range(HQ):
                rro = pltpu.make_async_remote_copy(
                    src_ref=rs_o.at[s_, :, h * DH:(h + 1) * DH],
                    dst_ref=rs_o.at[s_, :, h * DH:(h + 1) * DH],
                    send_sem=o_rsem.at[s_, h], recv_sem=o_rsem.at[s_, h],
                    device_id=(0,), device_id_type=pl.DeviceIdType.MESH)

                @pl.when(s_ != my)
                def _(rro=rro):
                    rro.wait_recv()

            rrl = pltpu.make_async_remote_copy(
                src_ref=rs_l.at[s_], dst_ref=rs_l.at[s_],
                send_sem=l_rsem.at[s_], recv_sem=l_rsem.at[s_],
                device_id=(0,), device_id_type=pl.DeviceIdType.MESH)

            @pl.when(s_ != my)
            def _(rrl=rrl):
                rrl.wait_recv()

            skip = s_ == my
            o_acc = o_acc + jnp.where(skip, 0.0, rs_o[s_].astype(jnp.float32))
            l_acc = l_acc + jnp.where(skip, 0.0, rs_l[s_])

        for h in range(HQ):
            y_div[:, h * DH:(h + 1) * DH] = (
                o_acc[:, h * DH:(h + 1) * DH] / l_acc[:, h:h + 1])
        y = jnp.dot(y_div[...].astype(jnp.bfloat16),
                    wo_ref[...].astype(jnp.bfloat16),
                    preferred_element_type=jnp.float32)
        out_ref[pl.ds(my * CHUNK, CHUNK), :] = y
        y_snd[...] = y.astype(jnp.bfloat16)

        rd_g = [None] * N_DEV
        for pid in range(N_DEV):
            rd_g[pid] = pltpu.make_async_remote_copy(
                src_ref=y_snd, dst_ref=ag_ref.at[my],
                send_sem=g_ssem.at[pid], recv_sem=g_rsem.at[my],
                device_id=(pid,), device_id_type=pl.DeviceIdType.MESH)

            @pl.when(pid != my)
            def _(pid=pid):
                rd_g[pid].start()

        for s_ in range(N_DEV):
            rrg = pltpu.make_async_remote_copy(
                src_ref=ag_ref.at[s_], dst_ref=ag_ref.at[s_],
                send_sem=g_rsem.at[s_], recv_sem=g_rsem.at[s_],
                device_id=(0,), device_id_type=pl.DeviceIdType.MESH)

            @pl.when(s_ != my)
            def _(rrg=rrg, s_=s_):
                rrg.wait_recv()
                out_ref[pl.ds(s_ * CHUNK, CHUNK), :] = (
                    ag_ref[s_].astype(jnp.float32))

        for pid in range(N_DEV):
            @pl.when(pid != my)
            def _(pid=pid):
                for h in range(HQ):
                    rd_o[pid][h].wait_send()
                rd_l[pid].wait_send()
                rd_g[pid].wait_send()

    out = pl.pallas_call(
        body,
        out_shape=jax.ShapeDtypeStruct((SQ, D), jnp.float32),
        in_specs=[pl.BlockSpec(memory_space=pltpu.VMEM)] * 5,
        out_specs=pl.BlockSpec(memory_space=pltpu.VMEM),
        scratch_shapes=[
            pltpu.VMEM((SQ, D), jnp.bfloat16),
            pltpu.VMEM((SQ, HQ), jnp.float32),
            pltpu.VMEM((N_DEV, CHUNK, D), jnp.bfloat16),
            pltpu.VMEM((N_DEV, CHUNK, HQ), jnp.float32),
            pltpu.VMEM((CHUNK, D), jnp.float32),
            pltpu.VMEM((CHUNK, D), jnp.bfloat16),
            pltpu.VMEM((N_DEV, CHUNK, D), jnp.bfloat16),
            pltpu.SemaphoreType.DMA((N_DEV, HQ)),
            pltpu.SemaphoreType.DMA((N_DEV, HQ)),
            pltpu.SemaphoreType.DMA((N_DEV,)),
            pltpu.SemaphoreType.DMA((N_DEV,)),
            pltpu.SemaphoreType.DMA((N_DEV,)),
            pltpu.SemaphoreType.DMA((N_DEV,)),
        ],
        compiler_params=pltpu.CompilerParams(collective_id=0),
    )(x2, Wq, Wo, k2, v2)
    return out.reshape(B, SQ, D)
